# Initial kernel scaffold; baseline (speedup 1.0000x reference)
import jax
import jax.numpy as jnp
from jax.experimental import pallas as pl


def kernel(input_feature, edge_index, adj_values, weight, bias):
    raise NotImplementedError("write your pallas kernel here")


# speedup vs baseline: 2.5901x; 2.5901x over previous
import functools

import jax
import jax.numpy as jnp
from jax import lax
from jax.experimental import pallas as pl
from jax.experimental.pallas import tpu as pltpu
from jax.experimental.pallas import tpu_sc as plsc

N = 10000
D = 128
E = 320000
NC = 2
NS = 16
NW = NC * NS
CH = 128
CPT = 80
EPT = CH * CPT
E_PAD = NW * EPT
N_PAD = 10240
RPT = N_PAD // NS
ZR = 128


def _sc_spmm(x, src, dst, val):
    mesh = plsc.VectorSubcoreMesh(core_axis_name="c", subcore_axis_name="s")

    @functools.partial(
        pl.kernel,
        out_type=jax.ShapeDtypeStruct((NC, N_PAD, D), jnp.float32),
        mesh=mesh,
        scratch_types=[
            pltpu.VMEM((CH,), jnp.int32),
            pltpu.VMEM((CH,), jnp.int32),
            pltpu.VMEM((CH,), jnp.float32),
            pltpu.VMEM((CH, D), jnp.float32),
            pltpu.VMEM((ZR, D), jnp.float32),
            pltpu.VMEM_SHARED((N_PAD, D), jnp.float32),
            pltpu.SemaphoreType.DMA,
        ],
    )
    def k(x_hbm, src_hbm, dst_hbm, val_hbm, out_hbm,
          src_v, dst_v, val_v, rows_v, zbuf, acc, sem):
        c = lax.axis_index("c")
        s = lax.axis_index("s")
        wid = c * NS + s

        zero = jnp.zeros((16,), jnp.float32)

        def zrow(r, _):
            for g in range(D // 16):
                zbuf[r, pl.ds(g * 16, 16)] = zero
            return 0

        lax.fori_loop(0, ZR, zrow, 0)

        row0 = s * RPT
        for kk in range(RPT // ZR):
            pltpu.sync_copy(zbuf, acc.at[pl.ds(row0 + kk * ZR, ZR)])
        plsc.subcore_barrier()

        ebase = wid * EPT

        dnums = lax.GatherDimensionNumbers(
            offset_dims=(), collapsed_slice_dims=(0,), start_index_map=(0,))
        jidx = [jnp.full((16, 1), j, jnp.int32) for j in range(16)]

        def scale_group(gi, _):
            eb = pl.multiple_of(gi * 16, 16)
            vv = val_v[pl.ds(eb, 16)]
            for j in range(16):
                vj = lax.gather(
                    vv, jidx[j], dnums, (1,),
                    mode=lax.GatherScatterMode.PROMISE_IN_BOUNDS)
                for g in range(D // 16):
                    sl = pl.ds(g * 16, 16)
                    rows_v[eb + j, sl] = rows_v[eb + j, sl] * vj
            return 0

        def chunk(ci, _):
            base = pl.multiple_of(ebase + ci * CH, CH)
            pltpu.sync_copy(src_hbm.at[pl.ds(base, CH)], src_v)
            pltpu.sync_copy(dst_hbm.at[pl.ds(base, CH)], dst_v)
            pltpu.sync_copy(val_hbm.at[pl.ds(base, CH)], val_v)
            pltpu.async_copy(x_hbm.at[src_v], rows_v, sem).wait()
            lax.fori_loop(0, CH // 16, scale_group, 0)
            pltpu.sync_copy(rows_v, acc.at[dst_v], add=True)
            return 0

        lax.fori_loop(0, CPT, chunk, 0)
        plsc.subcore_barrier()

        for kk in range(RPT // ZR):
            r = row0 + kk * ZR
            pltpu.sync_copy(acc.at[pl.ds(r, ZR)], zbuf)
            pltpu.sync_copy(zbuf, out_hbm.at[c, pl.ds(r, ZR)])

    return k(x, src, dst, val)


_BLK = 1000


def _tc_body(p_ref, w_ref, b_ref, o_ref):
    ssum = p_ref[0] + p_ref[1]
    o_ref[...] = (
        jnp.dot(ssum, w_ref[...], preferred_element_type=jnp.float32)
        + b_ref[...]
    )


def _tc_combine(partials, weight, bias):
    return pl.pallas_call(
        _tc_body,
        grid=(N // _BLK,),
        in_specs=[
            pl.BlockSpec((NC, _BLK, D), lambda i: (0, i, 0)),
            pl.BlockSpec((D, D), lambda i: (0, 0)),
            pl.BlockSpec((1, D), lambda i: (0, 0)),
        ],
        out_specs=pl.BlockSpec((_BLK, D), lambda i: (i, 0)),
        out_shape=jax.ShapeDtypeStruct((N, D), jnp.float32),
    )(partials, weight, bias.reshape(1, D))


@jax.jit
def kernel(input_feature, edge_index, adj_values, weight, bias):
    pad = E_PAD - E
    src = jnp.concatenate([edge_index[0], jnp.zeros((pad,), jnp.int32)])
    dst = jnp.concatenate([edge_index[1], jnp.zeros((pad,), jnp.int32)])
    val = jnp.concatenate([adj_values, jnp.zeros((pad,), jnp.float32)])
    partials = _sc_spmm(input_feature, src, dst, val)
    return _tc_combine(partials, weight, bias)
